# baseline (speedup 1.0000x reference)
import functools

import jax
import jax.numpy as jnp
from jax import lax
from jax.experimental import pallas as pl
from jax.experimental.pallas import tpu as pltpu
from jax.experimental.pallas import tpu_sc as plsc

N = 10000
NP = 10240
E = 640000
NEG = 0.2
BN_INV = 1.0 / (1.0 + 1e-5) ** 0.5
BN_B = 640
NB = NP // BN_B
NT = 16
RPT = NP // NT
EB = 64
CH = 8
ER = 10240
AR4 = 632
RWT4 = ER // NT
RWT1 = ER // (2 * NT)
NCH4 = RWT4 // CH
NCH1 = RWT1 // CH
F32 = jnp.float32

_HIGH = jax.lax.Precision.HIGHEST


def _dot(a, b):
    return jax.lax.dot_general(a, b, (((1,), (0,)), ((), ())),
                               preferred_element_type=F32, precision=_HIGH)


def _elu(v):
    return jnp.where(v > 0, v, jnp.exp(v) - 1.0)


def _tc1_body(x_ref, w_ref, asrc_ref, adst_ref, h_ref, als_ref, ald_ref):
    x = x_ref[...]
    for h in range(4):
        hh = _dot(x, w_ref[h])
        h_ref[h] = hh
        als_ref[h] = jnp.sum(hh * asrc_ref[h], axis=1)
        ald_ref[h] = jnp.sum(hh * adst_ref[h], axis=1)


def _tc1(x, w, asrc, adst):
    return pl.pallas_call(
        _tc1_body,
        grid=(NB,),
        in_specs=[
            pl.BlockSpec((BN_B, 128), lambda i: (i, 0)),
            pl.BlockSpec((4, 128, 128), lambda i: (0, 0, 0)),
            pl.BlockSpec((4, 128), lambda i: (0, 0)),
            pl.BlockSpec((4, 128), lambda i: (0, 0)),
        ],
        out_specs=[
            pl.BlockSpec((4, BN_B, 128), lambda i: (0, i, 0)),
            pl.BlockSpec((4, BN_B), lambda i: (0, i)),
            pl.BlockSpec((4, BN_B), lambda i: (0, i)),
        ],
        out_shape=[
            jax.ShapeDtypeStruct((4, NP, 128), F32),
            jax.ShapeDtypeStruct((4, NP), F32),
            jax.ShapeDtypeStruct((4, NP), F32),
        ],
    )(x, w, asrc, adst)


def _tc2_body(agg_ref, den_ref, b_ref, g_ref, be_ref, w_ref, asrc_ref,
              adst_ref, h2_ref, als_ref, ald_ref):
    x2 = []
    for h in range(4):
        d = jnp.sum(den_ref[h], axis=0)
        v = agg_ref[h] / (d[:, None] + 1e-16) + b_ref[h]
        v = v * (g_ref[h] * BN_INV) + be_ref[h]
        x2.append(_elu(v))
    for ho in range(4):
        acc = _dot(x2[0], w_ref[0, ho])
        for hi in range(1, 4):
            acc = acc + _dot(x2[hi], w_ref[hi, ho])
        h2_ref[ho] = acc
        als_ref[ho] = jnp.sum(acc * asrc_ref[ho], axis=1)
        ald_ref[ho] = jnp.sum(acc * adst_ref[ho], axis=1)


def _tc2(agg, den, b, g, be, w, asrc, adst):
    return pl.pallas_call(
        _tc2_body,
        grid=(NB,),
        in_specs=[
            pl.BlockSpec((4, BN_B, 128), lambda i: (0, i, 0)),
            pl.BlockSpec((4, NT, BN_B), lambda i: (0, 0, i)),
            pl.BlockSpec((4, 128), lambda i: (0, 0)),
            pl.BlockSpec((4, 128), lambda i: (0, 0)),
            pl.BlockSpec((4, 128), lambda i: (0, 0)),
            pl.BlockSpec((4, 4, 128, 128), lambda i: (0, 0, 0, 0)),
            pl.BlockSpec((4, 128), lambda i: (0, 0)),
            pl.BlockSpec((4, 128), lambda i: (0, 0)),
        ],
        out_specs=[
            pl.BlockSpec((4, BN_B, 128), lambda i: (0, i, 0)),
            pl.BlockSpec((4, BN_B), lambda i: (0, i)),
            pl.BlockSpec((4, BN_B), lambda i: (0, i)),
        ],
        out_shape=[
            jax.ShapeDtypeStruct((4, NP, 128), F32),
            jax.ShapeDtypeStruct((4, NP), F32),
            jax.ShapeDtypeStruct((4, NP), F32),
        ],
    )(agg, den, b, g, be, w, asrc, adst)


def _tc3_body(agg_ref, den_ref, b_ref, g_ref, be_ref, w_ref, asrc_ref,
              adst_ref, h3_ref, als_ref, ald_ref):
    acc = None
    for h in range(4):
        d = jnp.sum(den_ref[h], axis=0)
        v = agg_ref[h] / (d[:, None] + 1e-16) + b_ref[h]
        v = v * (g_ref[h] * BN_INV) + be_ref[h]
        v = _elu(v)
        p = _dot(v, w_ref[h])
        acc = p if acc is None else acc + p
    h3_ref[...] = acc
    als_ref[0] = jnp.sum(acc * asrc_ref[...], axis=1)
    ald_ref[0] = jnp.sum(acc * adst_ref[...], axis=1)


def _tc3(agg, den, b, g, be, w, asrc, adst):
    return pl.pallas_call(
        _tc3_body,
        grid=(NB,),
        in_specs=[
            pl.BlockSpec((4, BN_B, 128), lambda i: (0, i, 0)),
            pl.BlockSpec((4, NT, BN_B), lambda i: (0, 0, i)),
            pl.BlockSpec((4, 128), lambda i: (0, 0)),
            pl.BlockSpec((4, 128), lambda i: (0, 0)),
            pl.BlockSpec((4, 128), lambda i: (0, 0)),
            pl.BlockSpec((4, 128, 64), lambda i: (0, 0, 0)),
            pl.BlockSpec((1, 64), lambda i: (0, 0)),
            pl.BlockSpec((1, 64), lambda i: (0, 0)),
        ],
        out_specs=[
            pl.BlockSpec((BN_B, 64), lambda i: (i, 0)),
            pl.BlockSpec((1, BN_B), lambda i: (0, i)),
            pl.BlockSpec((1, BN_B), lambda i: (0, i)),
        ],
        out_shape=[
            jax.ShapeDtypeStruct((NP, 64), F32),
            jax.ShapeDtypeStruct((1, NP), F32),
            jax.ShapeDtypeStruct((1, NP), F32),
        ],
    )(agg, den, b, g, be, w, asrc, adst)


def _tc4_body(agg_ref, den_ref, b_ref, g_ref, be_ref, out_ref):
    d = jnp.sum(den_ref[0], axis=0) + jnp.sum(den_ref[1], axis=0)
    v = (agg_ref[0] + agg_ref[1]) / (d[:, None] + 1e-16) + b_ref[...]
    out_ref[...] = v * (g_ref[...] * BN_INV) + be_ref[...]


def _tc4(agg, den, b, g, be):
    return pl.pallas_call(
        _tc4_body,
        grid=(NB,),
        in_specs=[
            pl.BlockSpec((2, BN_B, 64), lambda i: (0, i, 0)),
            pl.BlockSpec((2, NT, BN_B), lambda i: (0, 0, i)),
            pl.BlockSpec((1, 64), lambda i: (0, 0)),
            pl.BlockSpec((1, 64), lambda i: (0, 0)),
            pl.BlockSpec((1, 64), lambda i: (0, 0)),
        ],
        out_specs=pl.BlockSpec((BN_B, 64), lambda i: (i, 0)),
        out_shape=jax.ShapeDtypeStruct((NP, 64), F32),
    )(agg, den, b, g, be)


_MESH = plsc.VectorSubcoreMesh(core_axis_name="c", subcore_axis_name="s",
                               num_cores=2, num_subcores=NT)


def _edge_round(head, table, als_hbm, ald_hbm, src2_hbm, dst2_hbm, agg_row,
                den_row, zrows_hbm, zden_hbm, acc_sh, als_v, ald_v, den_v,
                srcb_v, dstb_v, idxs, ws, rows, sems, s, row_base, nch,
                nvec, tab_off, arpt):
    pltpu.sync_copy(zrows_hbm, acc_sh.at[pl.ds(s * arpt, arpt)])
    pltpu.sync_copy(als_hbm.at[head], als_v)
    pltpu.sync_copy(ald_hbm.at[head], ald_v)
    pltpu.sync_copy(zden_hbm, den_v)
    plsc.subcore_barrier()
    iota = lax.iota(jnp.int32, 16)

    def launch_gather(bb, q):
        def iloop(j, _):
            sl = pl.ds(j * 16, 16)
            idxs[q][sl] = srcb_v[bb, sl] + tab_off
            return 0
        lax.fori_loop(0, EB // 16, iloop, 0)
        return pltpu.async_copy(table.at[idxs[q]], rows[q], sems[q])

    def process(bb, q, crow, gdesc):
        def wstep(j, _):
            sl = pl.ds(j * 16, 16)
            si = srcb_v[bb, sl]
            di = dstb_v[bb, sl]
            e = plsc.load_gather(als_v, [si]) + plsc.load_gather(ald_v, [di])
            e = jnp.where(e > 0, e, NEG * e)
            w = jnp.exp(e)
            gpos = (crow + bb) * EB + j * 16 + iota
            w = jnp.where(gpos < E, w, 0.0)
            ws[q][sl] = w
            plsc.addupdate_scatter(den_v, [di], w)
            return 0

        lax.fori_loop(0, EB // 16, wstep, 0)
        gdesc.wait()

        @plsc.parallel_loop(0, EB, 1, unroll=2)
        def scale(j):
            wj = plsc.load_gather(ws[q], [jnp.zeros((16,), jnp.int32) + j])
            for k in range(nvec):
                sk = pl.ds(k * 16, 16)
                rows[q][j, sk] = rows[q][j, sk] * wj

        pltpu.sync_copy(rows[q], acc_sh.at[dstb_v.at[bb]], add=True)

    def chunk(c, _):
        crow = row_base + c * CH
        pltpu.sync_copy(src2_hbm.at[pl.ds(crow, CH)], srcb_v)
        pltpu.sync_copy(dst2_hbm.at[pl.ds(crow, CH)], dstb_v)

        def pair(p2, _):
            b0 = p2 * 2
            g0 = launch_gather(b0, 0)
            g1 = launch_gather(b0 + 1, 1)
            process(b0, 0, crow, g0)
            process(b0 + 1, 1, crow, g1)
            return 0

        lax.fori_loop(0, CH // 2, pair, 0)
        return 0

    lax.fori_loop(0, nch, chunk, 0)
    plsc.subcore_barrier()
    pltpu.sync_copy(acc_sh.at[pl.ds(s * arpt, arpt)],
                    agg_row.at[pl.ds(s * arpt, arpt)])
    pltpu.sync_copy(den_v, den_row.at[s])
    plsc.subcore_barrier()


def _sc4_body(table, als_hbm, ald_hbm, src2_hbm, dst2_hbm, zrows_hbm,
              zden_hbm, agg_out, den_out, acc_sh, als_v, ald_v, den_v,
              srcb_v, dstb_v, idx0, idx1, w0, w1, rows0, rows1, sem0, sem1):
    c = lax.axis_index("c")
    s = lax.axis_index("s")

    def rnd(r, _):
        head = c * 2 + r
        _edge_round(head, table, als_hbm, ald_hbm, src2_hbm, dst2_hbm,
                    agg_out.at[head], den_out.at[head], zrows_hbm, zden_hbm,
                    acc_sh, als_v, ald_v, den_v, srcb_v, dstb_v,
                    (idx0, idx1), (w0, w1), (rows0, rows1), (sem0, sem1),
                    s, s * RWT4, NCH4, 8, head * NP, AR4)
        return 0

    lax.fori_loop(0, 2, rnd, 0)


def _sc4(table, als, ald, src2, dst2, zrows, zden):
    return pl.kernel(
        _sc4_body,
        out_type=(jax.ShapeDtypeStruct((4, NP, 128), F32),
                  jax.ShapeDtypeStruct((4, NT, NP), F32)),
        mesh=_MESH,
        compiler_params=pltpu.CompilerParams(needs_layout_passes=False, use_tc_tiling_on_sc=False),
        scratch_types=[
            pltpu.VMEM_SHARED((AR4 * NT, 128), F32),
            pltpu.VMEM((NP,), F32),
            pltpu.VMEM((NP,), F32),
            pltpu.VMEM((NP,), F32),
            pltpu.VMEM((CH, EB), jnp.int32),
            pltpu.VMEM((CH, EB), jnp.int32),
            pltpu.VMEM((EB,), jnp.int32),
            pltpu.VMEM((EB,), jnp.int32),
            pltpu.VMEM((EB,), F32),
            pltpu.VMEM((EB,), F32),
            pltpu.VMEM((EB, 128), F32),
            pltpu.VMEM((EB, 128), F32),
            pltpu.SemaphoreType.DMA,
            pltpu.SemaphoreType.DMA,
        ],
    )(table, als, ald, src2, dst2, zrows, zden)


def _sc1_body(table, als_hbm, ald_hbm, src2_hbm, dst2_hbm, zrows_hbm,
              zden_hbm, agg_out, den_out, acc_sh, als_v, ald_v, den_v,
              srcb_v, dstb_v, idx0, idx1, w0, w1, rows0, rows1, sem0, sem1):
    c = lax.axis_index("c")
    s = lax.axis_index("s")
    _edge_round(0, table, als_hbm, ald_hbm, src2_hbm, dst2_hbm, agg_out.at[c],
                den_out.at[c], zrows_hbm, zden_hbm, acc_sh, als_v, ald_v,
                den_v, srcb_v, dstb_v, (idx0, idx1), (w0, w1), (rows0, rows1),
                (sem0, sem1), s, (c * NT + s) * RWT1, NCH1, 4, 0, RPT)


def _sc1(table, als, ald, src2, dst2, zrows, zden):
    return pl.kernel(
        _sc1_body,
        out_type=(jax.ShapeDtypeStruct((2, NP, 64), F32),
                  jax.ShapeDtypeStruct((2, NT, NP), F32)),
        mesh=_MESH,
        compiler_params=pltpu.CompilerParams(needs_layout_passes=False, use_tc_tiling_on_sc=False),
        scratch_types=[
            pltpu.VMEM_SHARED((NP, 64), F32),
            pltpu.VMEM((NP,), F32),
            pltpu.VMEM((NP,), F32),
            pltpu.VMEM((NP,), F32),
            pltpu.VMEM((CH, EB), jnp.int32),
            pltpu.VMEM((CH, EB), jnp.int32),
            pltpu.VMEM((EB,), jnp.int32),
            pltpu.VMEM((EB,), jnp.int32),
            pltpu.VMEM((EB,), F32),
            pltpu.VMEM((EB,), F32),
            pltpu.VMEM((EB, 64), F32),
            pltpu.VMEM((EB, 64), F32),
            pltpu.SemaphoreType.DMA,
            pltpu.SemaphoreType.DMA,
        ],
    )(table, als, ald, src2, dst2, zrows, zden)


def kernel(x, edge_index, W1, a_src1, a_dst1, b1, g1, be1, W2, a_src2,
           a_dst2, b2, g2, be2, W3, a_src3, a_dst3, b3, g3, be3):
    epad = ER * EB - E
    src2 = jnp.pad(edge_index[0], (0, epad)).reshape(ER, EB)
    dst2 = jnp.pad(edge_index[1], (0, epad)).reshape(ER, EB)
    xp = jnp.pad(x, ((0, NP - N), (0, 0)))

    w1r = jnp.transpose(W1.reshape(128, 4, 128), (1, 0, 2))
    w2r = jnp.transpose(W2.reshape(4, 128, 4, 128), (0, 2, 1, 3))
    w3r = W3.reshape(4, 128, 64)
    as1, ad1 = a_src1.reshape(4, 128), a_dst1.reshape(4, 128)
    as2, ad2 = a_src2.reshape(4, 128), a_dst2.reshape(4, 128)
    as3, ad3 = a_src3.reshape(1, 64), a_dst3.reshape(1, 64)
    b1r, g1r, be1r = b1.reshape(4, 128), g1.reshape(4, 128), be1.reshape(4, 128)
    b2r, g2r, be2r = b2.reshape(4, 128), g2.reshape(4, 128), be2.reshape(4, 128)
    b3r, g3r, be3r = b3.reshape(1, 64), g3.reshape(1, 64), be3.reshape(1, 64)

    zden = jnp.zeros((NP,), F32)
    zrows = jnp.zeros((AR4, 128), F32)
    zrows3 = jnp.zeros((RPT, 64), F32)

    h1, als1, ald1 = _tc1(xp, w1r, as1, ad1)
    agg1, den1 = _sc4(h1.reshape(4 * NP, 128), als1, ald1, src2, dst2,
                      zrows, zden)
    h2, als2, ald2 = _tc2(agg1, den1, b1r, g1r, be1r, w2r, as2, ad2)
    agg2, den2 = _sc4(h2.reshape(4 * NP, 128), als2, ald2, src2, dst2,
                      zrows, zden)
    h3, als3, ald3 = _tc3(agg2, den2, b2r, g2r, be2r, w3r, as3, ad3)
    agg3, den3 = _sc1(h3, als3, ald3, src2, dst2, zrows3, zden)
    outp = _tc4(agg3, den3, b3r, g3r, be3r)
    return outp[:N]


# speedup vs baseline: 1.4117x; 1.4117x over previous
import functools

import jax
import jax.numpy as jnp
from jax import lax
from jax.experimental import pallas as pl
from jax.experimental.pallas import tpu as pltpu
from jax.experimental.pallas import tpu_sc as plsc

N = 10000
NP = 10240
E = 640000
NEG = 0.2
BN_INV = 1.0 / (1.0 + 1e-5) ** 0.5
BN_B = 640
NB = NP // BN_B
NT = 16
RPT = NP // NT
EB = 64
CH = 8
ER = 10240
AR4 = 632
RWT4 = ER // NT
RWT1 = ER // (2 * NT)
NQ4 = RWT4 // (4 * CH)
NQ1 = RWT1 // (4 * CH)
F32 = jnp.float32

_HIGH = jax.lax.Precision.HIGHEST


def _dot(a, b):
    return jax.lax.dot_general(a, b, (((1,), (0,)), ((), ())),
                               preferred_element_type=F32, precision=_HIGH)


def _elu(v):
    return jnp.where(v > 0, v, jnp.exp(v) - 1.0)


def _tc1_body(x_ref, w_ref, asrc_ref, adst_ref, h_ref, als_ref, ald_ref):
    x = x_ref[...]
    for h in range(4):
        hh = _dot(x, w_ref[h])
        h_ref[h] = hh
        als_ref[h] = jnp.sum(hh * asrc_ref[h], axis=1)
        ald_ref[h] = jnp.sum(hh * adst_ref[h], axis=1)


def _tc1(x, w, asrc, adst):
    return pl.pallas_call(
        _tc1_body,
        grid=(NB,),
        in_specs=[
            pl.BlockSpec((BN_B, 128), lambda i: (i, 0)),
            pl.BlockSpec((4, 128, 128), lambda i: (0, 0, 0)),
            pl.BlockSpec((4, 128), lambda i: (0, 0)),
            pl.BlockSpec((4, 128), lambda i: (0, 0)),
        ],
        out_specs=[
            pl.BlockSpec((4, BN_B, 128), lambda i: (0, i, 0)),
            pl.BlockSpec((4, BN_B), lambda i: (0, i)),
            pl.BlockSpec((4, BN_B), lambda i: (0, i)),
        ],
        out_shape=[
            jax.ShapeDtypeStruct((4, NP, 128), F32),
            jax.ShapeDtypeStruct((4, NP), F32),
            jax.ShapeDtypeStruct((4, NP), F32),
        ],
    )(x, w, asrc, adst)


def _tc2_body(agg_ref, den_ref, b_ref, g_ref, be_ref, w_ref, asrc_ref,
              adst_ref, h2_ref, als_ref, ald_ref):
    x2 = []
    for h in range(4):
        d = jnp.sum(den_ref[h], axis=0)
        v = agg_ref[h] / (d[:, None] + 1e-16) + b_ref[h]
        v = v * (g_ref[h] * BN_INV) + be_ref[h]
        x2.append(_elu(v))
    for ho in range(4):
        acc = _dot(x2[0], w_ref[0, ho])
        for hi in range(1, 4):
            acc = acc + _dot(x2[hi], w_ref[hi, ho])
        h2_ref[ho] = acc
        als_ref[ho] = jnp.sum(acc * asrc_ref[ho], axis=1)
        ald_ref[ho] = jnp.sum(acc * adst_ref[ho], axis=1)


def _tc2(agg, den, b, g, be, w, asrc, adst):
    return pl.pallas_call(
        _tc2_body,
        grid=(NB,),
        in_specs=[
            pl.BlockSpec((4, BN_B, 128), lambda i: (0, i, 0)),
            pl.BlockSpec((4, NT, BN_B), lambda i: (0, 0, i)),
            pl.BlockSpec((4, 128), lambda i: (0, 0)),
            pl.BlockSpec((4, 128), lambda i: (0, 0)),
            pl.BlockSpec((4, 128), lambda i: (0, 0)),
            pl.BlockSpec((4, 4, 128, 128), lambda i: (0, 0, 0, 0)),
            pl.BlockSpec((4, 128), lambda i: (0, 0)),
            pl.BlockSpec((4, 128), lambda i: (0, 0)),
        ],
        out_specs=[
            pl.BlockSpec((4, BN_B, 128), lambda i: (0, i, 0)),
            pl.BlockSpec((4, BN_B), lambda i: (0, i)),
            pl.BlockSpec((4, BN_B), lambda i: (0, i)),
        ],
        out_shape=[
            jax.ShapeDtypeStruct((4, NP, 128), F32),
            jax.ShapeDtypeStruct((4, NP), F32),
            jax.ShapeDtypeStruct((4, NP), F32),
        ],
    )(agg, den, b, g, be, w, asrc, adst)


def _tc3_body(agg_ref, den_ref, b_ref, g_ref, be_ref, w_ref, asrc_ref,
              adst_ref, h3_ref, als_ref, ald_ref):
    acc = None
    for h in range(4):
        d = jnp.sum(den_ref[h], axis=0)
        v = agg_ref[h] / (d[:, None] + 1e-16) + b_ref[h]
        v = v * (g_ref[h] * BN_INV) + be_ref[h]
        v = _elu(v)
        p = _dot(v, w_ref[h])
        acc = p if acc is None else acc + p
    h3_ref[...] = acc
    als_ref[0] = jnp.sum(acc * asrc_ref[...], axis=1)
    ald_ref[0] = jnp.sum(acc * adst_ref[...], axis=1)


def _tc3(agg, den, b, g, be, w, asrc, adst):
    return pl.pallas_call(
        _tc3_body,
        grid=(NB,),
        in_specs=[
            pl.BlockSpec((4, BN_B, 128), lambda i: (0, i, 0)),
            pl.BlockSpec((4, NT, BN_B), lambda i: (0, 0, i)),
            pl.BlockSpec((4, 128), lambda i: (0, 0)),
            pl.BlockSpec((4, 128), lambda i: (0, 0)),
            pl.BlockSpec((4, 128), lambda i: (0, 0)),
            pl.BlockSpec((4, 128, 64), lambda i: (0, 0, 0)),
            pl.BlockSpec((1, 64), lambda i: (0, 0)),
            pl.BlockSpec((1, 64), lambda i: (0, 0)),
        ],
        out_specs=[
            pl.BlockSpec((BN_B, 64), lambda i: (i, 0)),
            pl.BlockSpec((1, BN_B), lambda i: (0, i)),
            pl.BlockSpec((1, BN_B), lambda i: (0, i)),
        ],
        out_shape=[
            jax.ShapeDtypeStruct((NP, 64), F32),
            jax.ShapeDtypeStruct((1, NP), F32),
            jax.ShapeDtypeStruct((1, NP), F32),
        ],
    )(agg, den, b, g, be, w, asrc, adst)


def _tc4_body(agg_ref, den_ref, b_ref, g_ref, be_ref, out_ref):
    d = jnp.sum(den_ref[0], axis=0) + jnp.sum(den_ref[1], axis=0)
    v = (agg_ref[0] + agg_ref[1]) / (d[:, None] + 1e-16) + b_ref[...]
    out_ref[...] = v * (g_ref[...] * BN_INV) + be_ref[...]


def _tc4(agg, den, b, g, be):
    return pl.pallas_call(
        _tc4_body,
        grid=(NB,),
        in_specs=[
            pl.BlockSpec((2, BN_B, 64), lambda i: (0, i, 0)),
            pl.BlockSpec((2, NT, BN_B), lambda i: (0, 0, i)),
            pl.BlockSpec((1, 64), lambda i: (0, 0)),
            pl.BlockSpec((1, 64), lambda i: (0, 0)),
            pl.BlockSpec((1, 64), lambda i: (0, 0)),
        ],
        out_specs=pl.BlockSpec((BN_B, 64), lambda i: (i, 0)),
        out_shape=jax.ShapeDtypeStruct((NP, 64), F32),
    )(agg, den, b, g, be)


_MESH = plsc.VectorSubcoreMesh(core_axis_name="c", subcore_axis_name="s",
                               num_cores=2, num_subcores=NT)

RING = 4
NPAR = 4


def _edge_round(head, table, als_tab, ald_tab, src2_hbm, dst2_hbm, agg_row,
                den_row, zrows_hbm, zden_hbm, acc_sh, den_v,
                srcb, dstb, idxs, dixs, alsb, aldb, ws, rows, gsem, asem,
                ssem, chsem, s, row_base, nquad, nvec, tab_off, arpt):
    pltpu.sync_copy(zrows_hbm, acc_sh.at[pl.ds(s * arpt, arpt)])
    pltpu.sync_copy(zden_hbm, den_v)
    plsc.subcore_barrier()
    iota16 = lax.iota(jnp.int32, 16)
    nbat = nquad * NPAR * CH

    def chunk_load(c, par):
        crow = row_base + c * CH
        pltpu.async_copy(src2_hbm.at[pl.ds(crow, CH)], srcb[par], chsem[par])
        pltpu.async_copy(dst2_hbm.at[pl.ds(crow, CH)], dstb[par], chsem[par])

    def chunk_wait(par):
        pltpu.make_async_copy(src2_hbm.at[pl.ds(0, CH)], srcb[par],
                              chsem[par]).wait()
        pltpu.make_async_copy(dst2_hbm.at[pl.ds(0, CH)], dstb[par],
                              chsem[par]).wait()

    def launch_gather(q, par, bb):
        for j in range(EB // 16):
            sl = pl.ds(j * 16, 16)
            idxs[q][sl] = srcb[par][bb, sl] + tab_off
            dixs[q][sl] = dstb[par][bb, sl] + tab_off
        pltpu.async_copy(table.at[idxs[q]], rows[q], gsem[q])
        pltpu.async_copy(als_tab.at[idxs[q]], alsb[q], asem[q])
        pltpu.async_copy(ald_tab.at[dixs[q]], aldb[q], asem[q])

    def wait_scatter(q):
        pltpu.make_async_copy(rows[q], acc_sh.at[dstb[0].at[0]],
                              ssem[q]).wait()

    def process(q, par, bb, erow):
        pltpu.make_async_copy(als_tab.at[idxs[q]], alsb[q], asem[q]).wait()
        pltpu.make_async_copy(ald_tab.at[dixs[q]], aldb[q], asem[q]).wait()

        def wstep(j, _):
            sl = pl.ds(j * 16, 16)
            di = dstb[par][bb, sl]
            e = alsb[q][sl] + aldb[q][sl]
            e = jnp.where(e > 0, e, NEG * e)
            w = jnp.exp(e)
            gpos = erow * EB + j * 16 + iota16
            w = jnp.where(gpos < E, w, 0.0)
            ws[q][sl] = w
            plsc.addupdate_scatter(den_v, [di], w)
            return 0

        lax.fori_loop(0, EB // 16, wstep, 0)
        pltpu.make_async_copy(table.at[idxs[q]], rows[q], gsem[q]).wait()

        @plsc.parallel_loop(0, EB, 1, unroll=2)
        def scale(j):
            wj = plsc.load_gather(ws[q], [jnp.zeros((16,), jnp.int32) + j])
            for k in range(nvec):
                sk = pl.ds(k * 16, 16)
                rows[q][j, sk] = rows[q][j, sk] * wj

        pltpu.async_copy(rows[q], acc_sh.at[dstb[par].at[bb]], ssem[q],
                         add=True)

    chunk_load(0, 0)
    chunk_load(1, 1)

    def quad(c4, _):
        for cc in range(NPAR):
            for bb in range(CH):
                slot = cc * CH + bb
                if bb == 0:
                    chunk_wait(cc)
                    if cc < 2:
                        chunk_load(c4 * NPAR + cc + 2, cc + 2)
                    else:
                        @pl.when(c4 < nquad - 1)
                        def _():
                            chunk_load(c4 * NPAR + cc + 2, cc - 2)
                q = slot % RING
                if slot >= RING:
                    wait_scatter(q)
                else:
                    @pl.when(c4 > 0)
                    def _():
                        wait_scatter(q)
                launch_gather(q, cc, bb)
                ps = slot - 2
                erow_p = row_base + c4 * (NPAR * CH) + ps
                if ps >= 0:
                    process(ps % RING, ps // CH, ps % CH, erow_p)
                else:
                    pss = ps + NPAR * CH
                    @pl.when(c4 > 0)
                    def _():
                        process(pss % RING, pss // CH, pss % CH, erow_p)
        return 0

    lax.fori_loop(0, nquad, quad, 0)
    process((NPAR * CH - 2) % RING, NPAR - 1, CH - 2, row_base + nbat - 2)
    process((NPAR * CH - 1) % RING, NPAR - 1, CH - 1, row_base + nbat - 1)
    for q in range(RING):
        wait_scatter(q)
    plsc.subcore_barrier()
    pltpu.sync_copy(acc_sh.at[pl.ds(s * arpt, arpt)],
                    agg_row.at[pl.ds(s * arpt, arpt)])
    pltpu.sync_copy(den_v, den_row.at[s])
    plsc.subcore_barrier()


def _sc_scratch(width):
    return ([pltpu.VMEM((NP,), F32)]
            + [pltpu.VMEM((CH, EB), jnp.int32)] * (2 * NPAR)
            + [pltpu.VMEM((EB,), jnp.int32)] * (2 * RING)
            + [pltpu.VMEM((EB,), F32)] * (3 * RING)
            + [pltpu.VMEM((EB, width), F32)] * RING
            + [pltpu.SemaphoreType.DMA] * (3 * RING + NPAR))


def _sc_unpack(scr):
    den_v = scr[0]
    o = 1
    srcb = scr[o:o + NPAR]
    dstb = scr[o + NPAR:o + 2 * NPAR]
    o += 2 * NPAR
    idxs = scr[o:o + RING]
    dixs = scr[o + RING:o + 2 * RING]
    o += 2 * RING
    alsb = scr[o:o + RING]
    aldb = scr[o + RING:o + 2 * RING]
    ws = scr[o + 2 * RING:o + 3 * RING]
    o += 3 * RING
    rows = scr[o:o + RING]
    o += RING
    gsem = scr[o:o + RING]
    asem = scr[o + RING:o + 2 * RING]
    ssem = scr[o + 2 * RING:o + 3 * RING]
    chsem = scr[o + 3 * RING:o + 3 * RING + NPAR]
    return (den_v, srcb, dstb, idxs, dixs, alsb, aldb, ws, rows, gsem,
            asem, ssem, chsem)


def _sc4_body(table, als_tab, ald_tab, src2_hbm, dst2_hbm, zrows_hbm,
              zden_hbm, agg_out, den_out, acc_sh, *scr):
    c = lax.axis_index("c")
    s = lax.axis_index("s")
    (den_v, srcb, dstb, idxs, dixs, alsb, aldb, ws, rows, gsem, asem,
     ssem, chsem) = _sc_unpack(scr)

    def rnd(r, _):
        head = c * 2 + r
        _edge_round(head, table, als_tab, ald_tab, src2_hbm, dst2_hbm,
                    agg_out.at[head], den_out.at[head], zrows_hbm, zden_hbm,
                    acc_sh, den_v, srcb, dstb, idxs, dixs, alsb, aldb, ws,
                    rows, gsem, asem, ssem, chsem, s, s * RWT4, NQ4, 8,
                    head * NP, AR4)
        return 0

    lax.fori_loop(0, 2, rnd, 0)


def _sc4(table, als, ald, src2, dst2, zrows, zden):
    return pl.kernel(
        _sc4_body,
        out_type=(jax.ShapeDtypeStruct((4, NP, 128), F32),
                  jax.ShapeDtypeStruct((4, NT, NP), F32)),
        mesh=_MESH,
        compiler_params=pltpu.CompilerParams(needs_layout_passes=False, use_tc_tiling_on_sc=False),
        scratch_types=[pltpu.VMEM_SHARED((AR4 * NT, 128), F32)]
        + _sc_scratch(128),
    )(table, als, ald, src2, dst2, zrows, zden)


def _sc1_body(table, als_tab, ald_tab, src2_hbm, dst2_hbm, zrows_hbm,
              zden_hbm, agg_out, den_out, acc_sh, *scr):
    c = lax.axis_index("c")
    s = lax.axis_index("s")
    (den_v, srcb, dstb, idxs, dixs, alsb, aldb, ws, rows, gsem, asem,
     ssem, chsem) = _sc_unpack(scr)
    _edge_round(0, table, als_tab, ald_tab, src2_hbm, dst2_hbm, agg_out.at[c],
                den_out.at[c], zrows_hbm, zden_hbm, acc_sh, den_v, srcb,
                dstb, idxs, dixs, alsb, aldb, ws, rows, gsem, asem, ssem,
                chsem, s, (c * NT + s) * RWT1, NQ1, 4, 0, RPT)


def _sc1(table, als, ald, src2, dst2, zrows, zden):
    return pl.kernel(
        _sc1_body,
        out_type=(jax.ShapeDtypeStruct((2, NP, 64), F32),
                  jax.ShapeDtypeStruct((2, NT, NP), F32)),
        mesh=_MESH,
        compiler_params=pltpu.CompilerParams(needs_layout_passes=False, use_tc_tiling_on_sc=False),
        scratch_types=[pltpu.VMEM_SHARED((NP, 64), F32)] + _sc_scratch(64),
    )(table, als, ald, src2, dst2, zrows, zden)


def kernel(x, edge_index, W1, a_src1, a_dst1, b1, g1, be1, W2, a_src2,
           a_dst2, b2, g2, be2, W3, a_src3, a_dst3, b3, g3, be3):
    epad = ER * EB - E
    src2 = jnp.pad(edge_index[0], (0, epad)).reshape(ER, EB)
    dst2 = jnp.pad(edge_index[1], (0, epad)).reshape(ER, EB)
    xp = jnp.pad(x, ((0, NP - N), (0, 0)))

    w1r = jnp.transpose(W1.reshape(128, 4, 128), (1, 0, 2))
    w2r = jnp.transpose(W2.reshape(4, 128, 4, 128), (0, 2, 1, 3))
    w3r = W3.reshape(4, 128, 64)
    as1, ad1 = a_src1.reshape(4, 128), a_dst1.reshape(4, 128)
    as2, ad2 = a_src2.reshape(4, 128), a_dst2.reshape(4, 128)
    as3, ad3 = a_src3.reshape(1, 64), a_dst3.reshape(1, 64)
    b1r, g1r, be1r = b1.reshape(4, 128), g1.reshape(4, 128), be1.reshape(4, 128)
    b2r, g2r, be2r = b2.reshape(4, 128), g2.reshape(4, 128), be2.reshape(4, 128)
    b3r, g3r, be3r = b3.reshape(1, 64), g3.reshape(1, 64), be3.reshape(1, 64)

    zden = jnp.zeros((NP,), F32)
    zrows = jnp.zeros((AR4, 128), F32)
    zrows3 = jnp.zeros((RPT, 64), F32)

    h1, als1, ald1 = _tc1(xp, w1r, as1, ad1)
    agg1, den1 = _sc4(h1.reshape(4 * NP, 128), als1.reshape(-1),
                      ald1.reshape(-1), src2, dst2, zrows, zden)
    h2, als2, ald2 = _tc2(agg1, den1, b1r, g1r, be1r, w2r, as2, ad2)
    agg2, den2 = _sc4(h2.reshape(4 * NP, 128), als2.reshape(-1),
                      ald2.reshape(-1), src2, dst2, zrows, zden)
    h3, als3, ald3 = _tc3(agg2, den2, b2r, g2r, be2r, w3r, as3, ad3)
    agg3, den3 = _sc1(h3, als3.reshape(-1), ald3.reshape(-1), src2, dst2,
                      zrows3, zden)
    outp = _tc4(agg3, den3, b3r, g3r, be3r)
    return outp[:N]


# speedup vs baseline: 2.2327x; 1.5816x over previous
import functools

import jax
import jax.numpy as jnp
from jax import lax
from jax.experimental import pallas as pl
from jax.experimental.pallas import tpu as pltpu
from jax.experimental.pallas import tpu_sc as plsc

N = 10000
NP = 10240
E = 640000
NEG = 0.2
BN_INV = 1.0 / (1.0 + 1e-5) ** 0.5
BN_B = 640
NB = NP // BN_B
NT = 16
RPT = NP // NT
EB = 64
CH = 4
ER = 10240
AR4 = 625
RWT4 = ER // NT
RWT1 = ER // (2 * NT)
NQ4 = RWT4 // (4 * CH)
NQ1 = RWT1 // (4 * CH)
F32 = jnp.float32

_HIGH = jax.lax.Precision.HIGHEST


def _dot(a, b):
    return jax.lax.dot_general(a, b, (((1,), (0,)), ((), ())),
                               preferred_element_type=F32, precision=_HIGH)


def _elu(v):
    return jnp.where(v > 0, v, jnp.exp(v) - 1.0)


def _pack_half(v):
    hc = v.shape[-1] // 2
    bits = jax.lax.bitcast_convert_type(v.astype(jnp.bfloat16),
                                        jnp.uint16).astype(jnp.int32)
    return jnp.bitwise_or(jnp.left_shift(bits[:, hc:], 16), bits[:, :hc])


def _tc1_body(x_ref, w_ref, asrc_ref, adst_ref, h_ref, als_ref, ald_ref):
    x = x_ref[...]
    for h in range(4):
        hh = _dot(x, w_ref[h])
        h_ref[h] = _pack_half(hh)
        als_ref[h] = jnp.sum(hh * asrc_ref[h], axis=1)
        ald_ref[h] = jnp.sum(hh * adst_ref[h], axis=1)


def _tc1(x, w, asrc, adst):
    return pl.pallas_call(
        _tc1_body,
        grid=(NB,),
        in_specs=[
            pl.BlockSpec((BN_B, 128), lambda i: (i, 0)),
            pl.BlockSpec((4, 128, 128), lambda i: (0, 0, 0)),
            pl.BlockSpec((4, 128), lambda i: (0, 0)),
            pl.BlockSpec((4, 128), lambda i: (0, 0)),
        ],
        out_specs=[
            pl.BlockSpec((4, BN_B, 64), lambda i: (0, i, 0)),
            pl.BlockSpec((4, BN_B), lambda i: (0, i)),
            pl.BlockSpec((4, BN_B), lambda i: (0, i)),
        ],
        out_shape=[
            jax.ShapeDtypeStruct((4, NP, 64), jnp.int32),
            jax.ShapeDtypeStruct((4, NP), F32),
            jax.ShapeDtypeStruct((4, NP), F32),
        ],
    )(x, w, asrc, adst)


def _tc2_body(agg_ref, den_ref, b_ref, g_ref, be_ref, w_ref, asrc_ref,
              adst_ref, h2_ref, als_ref, ald_ref):
    x2 = []
    for h in range(4):
        d = jnp.sum(den_ref[h], axis=0)
        v = agg_ref[h] / (d[:, None] + 1e-16) + b_ref[h]
        v = v * (g_ref[h] * BN_INV) + be_ref[h]
        x2.append(_elu(v))
    for ho in range(4):
        acc = _dot(x2[0], w_ref[0, ho])
        for hi in range(1, 4):
            acc = acc + _dot(x2[hi], w_ref[hi, ho])
        h2_ref[ho] = _pack_half(acc)
        als_ref[ho] = jnp.sum(acc * asrc_ref[ho], axis=1)
        ald_ref[ho] = jnp.sum(acc * adst_ref[ho], axis=1)


def _tc2(agg, den, b, g, be, w, asrc, adst):
    return pl.pallas_call(
        _tc2_body,
        grid=(NB,),
        in_specs=[
            pl.BlockSpec((4, BN_B, 128), lambda i: (0, i, 0)),
            pl.BlockSpec((4, NT, BN_B), lambda i: (0, 0, i)),
            pl.BlockSpec((4, 128), lambda i: (0, 0)),
            pl.BlockSpec((4, 128), lambda i: (0, 0)),
            pl.BlockSpec((4, 128), lambda i: (0, 0)),
            pl.BlockSpec((4, 4, 128, 128), lambda i: (0, 0, 0, 0)),
            pl.BlockSpec((4, 128), lambda i: (0, 0)),
            pl.BlockSpec((4, 128), lambda i: (0, 0)),
        ],
        out_specs=[
            pl.BlockSpec((4, BN_B, 64), lambda i: (0, i, 0)),
            pl.BlockSpec((4, BN_B), lambda i: (0, i)),
            pl.BlockSpec((4, BN_B), lambda i: (0, i)),
        ],
        out_shape=[
            jax.ShapeDtypeStruct((4, NP, 64), jnp.int32),
            jax.ShapeDtypeStruct((4, NP), F32),
            jax.ShapeDtypeStruct((4, NP), F32),
        ],
    )(agg, den, b, g, be, w, asrc, adst)


def _tc3_body(agg_ref, den_ref, b_ref, g_ref, be_ref, w_ref, asrc_ref,
              adst_ref, h3_ref, als_ref, ald_ref):
    acc = None
    for h in range(4):
        d = jnp.sum(den_ref[h], axis=0)
        v = agg_ref[h] / (d[:, None] + 1e-16) + b_ref[h]
        v = v * (g_ref[h] * BN_INV) + be_ref[h]
        v = _elu(v)
        p = _dot(v, w_ref[h])
        acc = p if acc is None else acc + p
    h3_ref[...] = _pack_half(acc)
    als_ref[0] = jnp.sum(acc * asrc_ref[...], axis=1)
    ald_ref[0] = jnp.sum(acc * adst_ref[...], axis=1)


def _tc3(agg, den, b, g, be, w, asrc, adst):
    return pl.pallas_call(
        _tc3_body,
        grid=(NB,),
        in_specs=[
            pl.BlockSpec((4, BN_B, 128), lambda i: (0, i, 0)),
            pl.BlockSpec((4, NT, BN_B), lambda i: (0, 0, i)),
            pl.BlockSpec((4, 128), lambda i: (0, 0)),
            pl.BlockSpec((4, 128), lambda i: (0, 0)),
            pl.BlockSpec((4, 128), lambda i: (0, 0)),
            pl.BlockSpec((4, 128, 64), lambda i: (0, 0, 0)),
            pl.BlockSpec((1, 64), lambda i: (0, 0)),
            pl.BlockSpec((1, 64), lambda i: (0, 0)),
        ],
        out_specs=[
            pl.BlockSpec((BN_B, 32), lambda i: (i, 0)),
            pl.BlockSpec((1, BN_B), lambda i: (0, i)),
            pl.BlockSpec((1, BN_B), lambda i: (0, i)),
        ],
        out_shape=[
            jax.ShapeDtypeStruct((NP, 32), jnp.int32),
            jax.ShapeDtypeStruct((1, NP), F32),
            jax.ShapeDtypeStruct((1, NP), F32),
        ],
    )(agg, den, b, g, be, w, asrc, adst)


def _tc4_body(agg_ref, den_ref, b_ref, g_ref, be_ref, out_ref):
    d = jnp.sum(den_ref[0], axis=0) + jnp.sum(den_ref[1], axis=0)
    v = (agg_ref[0] + agg_ref[1]) / (d[:, None] + 1e-16) + b_ref[...]
    out_ref[...] = v * (g_ref[...] * BN_INV) + be_ref[...]


def _tc4(agg, den, b, g, be):
    return pl.pallas_call(
        _tc4_body,
        grid=(NB,),
        in_specs=[
            pl.BlockSpec((2, BN_B, 64), lambda i: (0, i, 0)),
            pl.BlockSpec((2, NT, BN_B), lambda i: (0, 0, i)),
            pl.BlockSpec((1, 64), lambda i: (0, 0)),
            pl.BlockSpec((1, 64), lambda i: (0, 0)),
            pl.BlockSpec((1, 64), lambda i: (0, 0)),
        ],
        out_specs=pl.BlockSpec((BN_B, 64), lambda i: (i, 0)),
        out_shape=jax.ShapeDtypeStruct((NP, 64), F32),
    )(agg, den, b, g, be)


_MESH = plsc.VectorSubcoreMesh(core_axis_name="c", subcore_axis_name="s",
                               num_cores=2, num_subcores=NT)

RING = 4
RS = 2
NPAR = 4


def _edge_round(head, table, als_tab, ald_tab, src2_hbm, dst2_hbm, agg_row,
                den_row, zrows_hbm, zden_hbm, acc_sh, den_v,
                srcb, dstb, idxs, dixs, alsb, aldb, ws, rows, rowsb, gsem,
                asem, ssem, chsem, s, row_base, nquad, hc, tab_off, arpt):
    pltpu.sync_copy(zrows_hbm, acc_sh.at[pl.ds(s * arpt, arpt)])
    pltpu.sync_copy(zden_hbm, den_v)
    plsc.subcore_barrier()
    iota16 = lax.iota(jnp.int32, 16)
    nbat = nquad * NPAR * CH

    def chunk_load(c, par):
        crow = row_base + c * CH
        pltpu.async_copy(src2_hbm.at[pl.ds(crow, CH)], srcb[par], chsem[par])
        pltpu.async_copy(dst2_hbm.at[pl.ds(crow, CH)], dstb[par], chsem[par])

    def chunk_wait(par):
        pltpu.make_async_copy(src2_hbm.at[pl.ds(0, CH)], srcb[par],
                              chsem[par]).wait()
        pltpu.make_async_copy(dst2_hbm.at[pl.ds(0, CH)], dstb[par],
                              chsem[par]).wait()

    def launch_gather(q, par, bb):
        for j in range(EB // 16):
            sl = pl.ds(j * 16, 16)
            idxs[q][sl] = srcb[par][bb, sl] + tab_off
            dixs[q][sl] = dstb[par][bb, sl] + tab_off
        pltpu.async_copy(table.at[idxs[q]], rowsb[q], gsem[q])
        pltpu.async_copy(als_tab.at[idxs[q]], alsb[q], asem[q])
        pltpu.async_copy(ald_tab.at[dixs[q]], aldb[q], asem[q])

    def wait_scatter(qr):
        pltpu.make_async_copy(rows[qr], acc_sh.at[dstb[0].at[0]],
                              ssem[qr]).wait()

    def process(q, qr, par, bb, erow):
        pltpu.make_async_copy(als_tab.at[idxs[q]], alsb[q], asem[q]).wait()
        pltpu.make_async_copy(ald_tab.at[dixs[q]], aldb[q], asem[q]).wait()

        def wstep(j, _):
            sl = pl.ds(j * 16, 16)
            di = dstb[par][bb, sl]
            e = alsb[q][sl] + aldb[q][sl]
            e = jnp.where(e > 0, e, NEG * e)
            w = jnp.exp(e)
            gpos = erow * EB + j * 16 + iota16
            w = jnp.where(gpos < E, w, 0.0)
            ws[q][sl] = w
            plsc.addupdate_scatter(den_v, [di], w)
            return 0

        lax.fori_loop(0, EB // 16, wstep, 0)
        pltpu.make_async_copy(table.at[idxs[q]], rowsb[q], gsem[q]).wait()

        @plsc.parallel_loop(0, EB, 1, unroll=2)
        def scale(j):
            wj = plsc.load_gather(ws[q], [jnp.zeros((16,), jnp.int32) + j])
            for k in range(hc // 16):
                sk = pl.ds(k * 16, 16)
                ab = plsc.bitcast(rowsb[q][j, sk], jnp.bfloat16)
                lo, hi = plsc.unpack(ab, format=plsc.PackFormat.INTERLEAVED)
                rows[qr][j, sk] = lo * wj
                rows[qr][j, pl.ds(hc + k * 16, 16)] = hi * wj

        pltpu.async_copy(rows[qr], acc_sh.at[dstb[par].at[bb]], ssem[qr],
                         add=True)

    chunk_load(0, 0)
    chunk_load(1, 1)

    def quad(c4, _):
        for cc in range(NPAR):
            for bb in range(CH):
                slot = cc * CH + bb
                if bb == 0:
                    chunk_wait(cc)
                    if cc < 2:
                        chunk_load(c4 * NPAR + cc + 2, cc + 2)
                    else:
                        @pl.when(c4 < nquad - 1)
                        def _():
                            chunk_load(c4 * NPAR + cc + 2, cc - 2)
                q = slot % RING
                launch_gather(q, cc, bb)
                ps = slot - 2
                erow_p = row_base + c4 * (NPAR * CH) + ps
                if ps >= 0:
                    if ps >= RS:
                        wait_scatter(ps % RS)
                        process(ps % RING, ps % RS, ps // CH, ps % CH,
                                erow_p)
                    else:
                        @pl.when(c4 > 0)
                        def _():
                            wait_scatter(ps % RS)
                        process(ps % RING, ps % RS, ps // CH, ps % CH,
                                erow_p)
                else:
                    pss = ps + NPAR * CH
                    @pl.when(c4 > 0)
                    def _():
                        wait_scatter(pss % RS)
                        process(pss % RING, pss % RS, pss // CH, pss % CH,
                                erow_p)
        return 0

    lax.fori_loop(0, nquad, quad, 0)
    ta, tb = NPAR * CH - 2, NPAR * CH - 1
    wait_scatter(ta % RS)
    process(ta % RING, ta % RS, ta // CH, ta % CH, row_base + nbat - 2)
    wait_scatter(tb % RS)
    process(tb % RING, tb % RS, tb // CH, tb % CH, row_base + nbat - 1)
    for qr in range(RS):
        wait_scatter(qr)
    plsc.subcore_barrier()
    pltpu.sync_copy(acc_sh.at[pl.ds(s * arpt, arpt)],
                    agg_row.at[pl.ds(s * arpt, arpt)])
    pltpu.sync_copy(den_v, den_row.at[s])
    plsc.subcore_barrier()


def _sc_scratch(width):
    return ([pltpu.VMEM((NP,), F32)]
            + [pltpu.VMEM((CH, EB), jnp.int32)] * (2 * NPAR)
            + [pltpu.VMEM((EB,), jnp.int32)] * (2 * RING)
            + [pltpu.VMEM((EB,), F32)] * (3 * RING)
            + [pltpu.VMEM((EB, width), F32)] * RS
            + [pltpu.VMEM((EB, width // 2), jnp.int32)] * RING
            + [pltpu.SemaphoreType.DMA] * (2 * RING + RS + NPAR))


def _sc_unpack(scr):
    den_v = scr[0]
    o = 1
    srcb = scr[o:o + NPAR]
    dstb = scr[o + NPAR:o + 2 * NPAR]
    o += 2 * NPAR
    idxs = scr[o:o + RING]
    dixs = scr[o + RING:o + 2 * RING]
    o += 2 * RING
    alsb = scr[o:o + RING]
    aldb = scr[o + RING:o + 2 * RING]
    ws = scr[o + 2 * RING:o + 3 * RING]
    o += 3 * RING
    rows = scr[o:o + RS]
    rowsb = scr[o + RS:o + RS + RING]
    o += RS + RING
    gsem = scr[o:o + RING]
    asem = scr[o + RING:o + 2 * RING]
    ssem = scr[o + 2 * RING:o + 2 * RING + RS]
    chsem = scr[o + 2 * RING + RS:o + 2 * RING + RS + NPAR]
    return (den_v, srcb, dstb, idxs, dixs, alsb, aldb, ws, rows, rowsb,
            gsem, asem, ssem, chsem)


def _sc4_body(table, als_tab, ald_tab, src2_hbm, dst2_hbm, zrows_hbm,
              zden_hbm, agg_out, den_out, acc_sh, *scr):
    c = lax.axis_index("c")
    s = lax.axis_index("s")
    (den_v, srcb, dstb, idxs, dixs, alsb, aldb, ws, rows, rowsb, gsem,
     asem, ssem, chsem) = _sc_unpack(scr)

    def rnd(r, _):
        head = c * 2 + r
        _edge_round(head, table, als_tab, ald_tab, src2_hbm, dst2_hbm,
                    agg_out.at[head], den_out.at[head], zrows_hbm, zden_hbm,
                    acc_sh, den_v, srcb, dstb, idxs, dixs, alsb, aldb, ws,
                    rows, rowsb, gsem, asem, ssem, chsem, s, s * RWT4, NQ4,
                    64, head * NP, AR4)
        return 0

    lax.fori_loop(0, 2, rnd, 0)


def _sc4(table, als, ald, src2, dst2, zrows, zden):
    return pl.kernel(
        _sc4_body,
        out_type=(jax.ShapeDtypeStruct((4, NP, 128), F32),
                  jax.ShapeDtypeStruct((4, NT, NP), F32)),
        mesh=_MESH,
        compiler_params=pltpu.CompilerParams(needs_layout_passes=False, use_tc_tiling_on_sc=False),
        scratch_types=[pltpu.VMEM_SHARED((AR4 * NT, 128), F32)]
        + _sc_scratch(128),
    )(table, als, ald, src2, dst2, zrows, zden)


def _sc1_body(table, als_tab, ald_tab, src2_hbm, dst2_hbm, zrows_hbm,
              zden_hbm, agg_out, den_out, acc_sh, *scr):
    c = lax.axis_index("c")
    s = lax.axis_index("s")
    (den_v, srcb, dstb, idxs, dixs, alsb, aldb, ws, rows, rowsb, gsem,
     asem, ssem, chsem) = _sc_unpack(scr)
    _edge_round(0, table, als_tab, ald_tab, src2_hbm, dst2_hbm, agg_out.at[c],
                den_out.at[c], zrows_hbm, zden_hbm, acc_sh, den_v, srcb,
                dstb, idxs, dixs, alsb, aldb, ws, rows, rowsb, gsem, asem,
                ssem, chsem, s, (c * NT + s) * RWT1, NQ1, 32, 0, RPT)


def _sc1(table, als, ald, src2, dst2, zrows, zden):
    return pl.kernel(
        _sc1_body,
        out_type=(jax.ShapeDtypeStruct((2, NP, 64), F32),
                  jax.ShapeDtypeStruct((2, NT, NP), F32)),
        mesh=_MESH,
        compiler_params=pltpu.CompilerParams(needs_layout_passes=False, use_tc_tiling_on_sc=False),
        scratch_types=[pltpu.VMEM_SHARED((NP, 64), F32)] + _sc_scratch(64),
    )(table, als, ald, src2, dst2, zrows, zden)


def kernel(x, edge_index, W1, a_src1, a_dst1, b1, g1, be1, W2, a_src2,
           a_dst2, b2, g2, be2, W3, a_src3, a_dst3, b3, g3, be3):
    epad = ER * EB - E
    src2 = jnp.pad(edge_index[0], (0, epad)).reshape(ER, EB)
    dst2 = jnp.pad(edge_index[1], (0, epad)).reshape(ER, EB)
    xp = jnp.pad(x, ((0, NP - N), (0, 0)))

    w1r = jnp.transpose(W1.reshape(128, 4, 128), (1, 0, 2))
    w2r = jnp.transpose(W2.reshape(4, 128, 4, 128), (0, 2, 1, 3))
    w3r = W3.reshape(4, 128, 64)
    as1, ad1 = a_src1.reshape(4, 128), a_dst1.reshape(4, 128)
    as2, ad2 = a_src2.reshape(4, 128), a_dst2.reshape(4, 128)
    as3, ad3 = a_src3.reshape(1, 64), a_dst3.reshape(1, 64)
    b1r, g1r, be1r = b1.reshape(4, 128), g1.reshape(4, 128), be1.reshape(4, 128)
    b2r, g2r, be2r = b2.reshape(4, 128), g2.reshape(4, 128), be2.reshape(4, 128)
    b3r, g3r, be3r = b3.reshape(1, 64), g3.reshape(1, 64), be3.reshape(1, 64)

    zden = jnp.zeros((NP,), F32)
    zrows = jnp.zeros((AR4, 128), F32)
    zrows3 = jnp.zeros((RPT, 64), F32)

    h1, als1, ald1 = _tc1(xp, w1r, as1, ad1)
    agg1, den1 = _sc4(h1.reshape(4 * NP, 64), als1.reshape(-1),
                      ald1.reshape(-1), src2, dst2, zrows, zden)
    h2, als2, ald2 = _tc2(agg1, den1, b1r, g1r, be1r, w2r, as2, ad2)
    agg2, den2 = _sc4(h2.reshape(4 * NP, 64), als2.reshape(-1),
                      ald2.reshape(-1), src2, dst2, zrows, zden)
    h3, als3, ald3 = _tc3(agg2, den2, b2r, g2r, be2r, w3r, as3, ad3)
    agg3, den3 = _sc1(h3, als3.reshape(-1), ald3.reshape(-1), src2, dst2,
                      zrows3, zden)
    outp = _tc4(agg3, den3, b3r, g3r, be3r)
    return outp[:N]
